# baseline (speedup 1.0000x reference)
import functools

import jax
import jax.numpy as jnp
from jax import lax
from jax.experimental import pallas as pl
from jax.experimental.pallas import tpu as pltpu
from jax.experimental.pallas import tpu_sc as plsc

N_NODES = 50000
N_EDGES = 1600000
IN_CH = 128
HID = 32
NUM_GRAPHS = 512
BN_EPS = 1e-5

NC = 2
NS = 16
NW = NC * NS
CH = 128
NBUF = 8
G = 24
NGRP = 34
STEPS_T = NGRP * G
E_PAD = STEPS_T * CH * NS
STEPS = E_PAD // NW // CH

NT = 1664
NPAD = NT * NW
GP = 520
DUMMY = N_NODES
WB = NPAD // NS


def _sc_deg_body(dst_hbm, out_hbm, dst_v, ones_v, zrow_v, deg_sh, dsem):
    c = lax.axis_index("c")
    s = lax.axis_index("s")
    w = c * NS + s
    pltpu.sync_copy(dst_hbm.at[w], dst_v)
    for i in range(CH // 16):
        ones_v[pl.ds(i * 16, 16)] = jnp.ones((16,), jnp.float32)

    def zbody(i, carry):
        zrow_v[pl.ds(i * 16, 16)] = jnp.zeros((16,), jnp.float32)
        return carry

    lax.fori_loop(0, WB // 16, zbody, 0)
    pltpu.sync_copy(zrow_v, deg_sh.at[pl.ds(s * WB, WB)])
    plsc.subcore_barrier()

    def body(q, carry):
        ds_ = [pltpu.async_copy(ones_v, deg_sh.at[dst_v.at[q * 8 + b]],
                                dsem, add=True) for b in range(8)]
        for d in ds_:
            d.wait()
        return carry

    lax.fori_loop(0, STEPS // 8, body, 0)
    plsc.subcore_barrier()
    pltpu.sync_copy(deg_sh.at[pl.ds(s * WB, WB)],
                    out_hbm.at[pl.ds(c * NPAD + s * WB, WB)])


_deg_call = pl.kernel(
    _sc_deg_body,
    out_type=jax.ShapeDtypeStruct((NC * NPAD,), jnp.float32),
    mesh=plsc.VectorSubcoreMesh(core_axis_name="c", subcore_axis_name="s"),
    scratch_types=[
        pltpu.VMEM((STEPS, CH), jnp.int32),
        pltpu.VMEM((CH,), jnp.float32),
        pltpu.VMEM((WB,), jnp.float32),
        pltpu.VMEM_SHARED((NPAD,), jnp.float32),
        pltpu.SemaphoreType.DMA,
    ],
    compiler_params=pltpu.CompilerParams(use_tc_tiling_on_sc=False),
)


HH = HID // 2


def _sc_main_body(src_hbm, dst_hbm, g_hbm, out_hbm, src_v, dst_v, rows_v,
                  agg_sh, g_sh, gsem, ssem):
    c = lax.axis_index("c")
    s = lax.axis_index("s")
    pltpu.sync_copy(g_hbm.at[c, pl.ds(s * WB, WB)], g_sh.at[pl.ds(s * WB, WB)])
    z16 = jnp.zeros((16,), jnp.float32)
    zbuf = rows_v.at[0]
    for r in range(CH):
        zbuf[r, 0:16] = z16
    base = s * WB
    for k in range(WB // CH):
        pltpu.sync_copy(zbuf, agg_sh.at[pl.ds(base + k * CH, CH)])
    plsc.subcore_barrier()

    def grp(gi, carry):
        pltpu.sync_copy(src_hbm.at[s, gi], src_v)
        pltpu.sync_copy(dst_hbm.at[s, gi], dst_v)

        def octet(q, carry2):
            t0 = q * NBUF
            gd, sd = [], []
            for b in range(NBUF):
                gd.append(pltpu.async_copy(
                    g_sh.at[src_v.at[t0 + b]], rows_v.at[b], gsem))
            for d in gd:
                d.wait()
            for b in range(NBUF):
                sd.append(pltpu.async_copy(
                    rows_v.at[b], agg_sh.at[dst_v.at[t0 + b]], ssem, add=True))
            for d in sd:
                d.wait()
            return carry2

        lax.fori_loop(0, G // NBUF, octet, 0)
        return carry

    lax.fori_loop(0, NGRP, grp, 0)
    plsc.subcore_barrier()
    pltpu.sync_copy(agg_sh.at[pl.ds(s * WB, WB)], out_hbm.at[c, pl.ds(s * WB, WB)])


_main_call = pl.kernel(
    _sc_main_body,
    out_type=jax.ShapeDtypeStruct((NC, NPAD, HH), jnp.float32),
    mesh=plsc.VectorSubcoreMesh(core_axis_name="c", subcore_axis_name="s"),
    scratch_types=[
        pltpu.VMEM((G, CH), jnp.int32),
        pltpu.VMEM((G, CH), jnp.int32),
        pltpu.VMEM((NBUF, CH, HH), jnp.float32),
        pltpu.VMEM_SHARED((NPAD, HH), jnp.float32),
        pltpu.VMEM_SHARED((NPAD, HH), jnp.float32),
        pltpu.SemaphoreType.DMA,
        pltpu.SemaphoreType.DMA,
    ],
    compiler_params=pltpu.CompilerParams(use_tc_tiling_on_sc=False),
)


def _sc_pool_body(h2_hbm, batch_hbm, ninf_hbm, out_hbm, h2_v, b_v, pool_v):
    c = lax.axis_index("c")
    s = lax.axis_index("s")
    w = c * NS + s
    base = w * NT
    pltpu.sync_copy(h2_hbm.at[pl.ds(base, NT)], h2_v)
    pltpu.sync_copy(batch_hbm.at[pl.ds(base, NT)], b_v)
    pltpu.sync_copy(ninf_hbm, pool_v)
    iota = lax.iota(jnp.int32, 16)

    def body(n, carry):
        nspl = jnp.full((16,), n, jnp.int32)
        b = plsc.load_gather(b_v, [nspl])
        for half in (0, 16):
            col = iota + half
            rowv = plsc.load_gather(h2_v, [nspl, col])
            cur = plsc.load_gather(pool_v, [b, col])
            plsc.store_scatter(pool_v, [b, col], jnp.maximum(cur, rowv))
        return carry

    lax.fori_loop(0, NT, body, 0)
    pltpu.sync_copy(pool_v, out_hbm.at[w])


_pool_call = pl.kernel(
    _sc_pool_body,
    out_type=jax.ShapeDtypeStruct((NW, GP, HID), jnp.float32),
    mesh=plsc.VectorSubcoreMesh(core_axis_name="c", subcore_axis_name="s"),
    scratch_types=[
        pltpu.VMEM((NT, HID), jnp.float32),
        pltpu.VMEM((NT,), jnp.int32),
        pltpu.VMEM((GP, HID), jnp.float32),
    ],
    compiler_params=pltpu.CompilerParams(use_tc_tiling_on_sc=False,
                                         needs_layout_passes=False),
)


BLK = 1664


def _tc_embed_body(x_ref, degp_ref, w1, b1, w2, b2, w3, b3, g_ref, dinv_ref):
    deg = degp_ref[0] + degp_ref[1]
    dinv = lax.rsqrt(jnp.maximum(deg, 1.0))
    h = jnp.maximum(jnp.dot(x_ref[...], w1[...],
                            preferred_element_type=jnp.float32) + b1[...], 0.0)
    h = jnp.maximum(jnp.dot(h, w2[...],
                            preferred_element_type=jnp.float32) + b2[...], 0.0)
    h = jnp.dot(h, w3[...], preferred_element_type=jnp.float32) + b3[...]
    g = h * dinv
    g_ref[0] = g[:, :HH]
    g_ref[1] = g[:, HH:]
    dinv_ref[...] = dinv


def _embed_call(xp, degp, W1, b1, W2, b2, W3, b3):
    full = lambda shape: pl.BlockSpec(shape, lambda i: (0,) * len(shape))
    return pl.pallas_call(
        _tc_embed_body,
        grid=(NPAD // BLK,),
        in_specs=[
            pl.BlockSpec((BLK, IN_CH), lambda i: (i, 0)),
            pl.BlockSpec((NC, BLK, 1), lambda i: (0, i, 0)),
            full((IN_CH, HID)), full((1, HID)),
            full((HID, HID)), full((1, HID)),
            full((HID, HID)), full((1, HID)),
        ],
        out_specs=[
            pl.BlockSpec((NC, BLK, HH), lambda i: (0, i, 0)),
            pl.BlockSpec((BLK, 1), lambda i: (i, 0)),
        ],
        out_shape=[
            jax.ShapeDtypeStruct((NC, NPAD, HH), jnp.float32),
            jax.ShapeDtypeStruct((NPAD, 1), jnp.float32),
        ],
    )(xp, degp, W1, b1.reshape(1, HID), W2, b2.reshape(1, HID),
      W3, b3.reshape(1, HID))


def _leaky(x):
    return jnp.where(x >= 0.0, x, 0.01 * x)


def _tc_gcn_body(aggp_ref, dinv_ref, w1, b1, w2, b2, h2_ref):
    a = jnp.concatenate([aggp_ref[0], aggp_ref[1]], axis=1) * dinv_ref[...]
    t = _leaky(jnp.dot(a, w1[...], preferred_element_type=jnp.float32) + b1[...])
    h2_ref[...] = _leaky(jnp.dot(t, w2[...],
                                 preferred_element_type=jnp.float32) + b2[...])


def _gcn_call(aggp, dinv, gW1, gb1, gW2, gb2):
    full = lambda shape: pl.BlockSpec(shape, lambda i: (0,) * len(shape))
    return pl.pallas_call(
        _tc_gcn_body,
        grid=(NPAD // BLK,),
        in_specs=[
            pl.BlockSpec((NC, BLK, HH), lambda i: (0, i, 0)),
            pl.BlockSpec((BLK, 1), lambda i: (i, 0)),
            full((HID, HID)), full((1, HID)),
            full((HID, HID)), full((1, HID)),
        ],
        out_specs=pl.BlockSpec((BLK, HID), lambda i: (i, 0)),
        out_shape=jax.ShapeDtypeStruct((NPAD, HID), jnp.float32),
    )(aggp, dinv, gW1, gb1.reshape(1, HID), gW2, gb2.reshape(1, HID))


def _tc_head_body(poolp_ref, mw1, mb1, gam, bet, mw2, mb2, out_ref, sig_ref):
    pm = jnp.max(poolp_ref[...], axis=0)[:NUM_GRAPHS]
    pooled = jnp.where(jnp.isfinite(pm), pm, 0.0)
    z = jnp.dot(pooled, mw1[...], preferred_element_type=jnp.float32) + mb1[...]
    z = (z / jnp.sqrt(1.0 + BN_EPS)) * gam[...] + bet[...]
    z = _leaky(z)
    o = jnp.dot(z, mw2[...], preferred_element_type=jnp.float32) + mb2[...]
    out_ref[...] = o
    sig_ref[...] = jax.nn.sigmoid(o)


def _head_call(poolp, mW1, mb1, bn_gamma, bn_beta, mW2, mb2):
    full = lambda shape: pl.BlockSpec(shape, lambda: (0,) * len(shape))
    return pl.pallas_call(
        _tc_head_body,
        in_specs=[
            full((NW, GP, HID)),
            full((HID, HID)), full((1, HID)),
            full((1, HID)), full((1, HID)),
            full((HID, 1)), full((1, 1)),
        ],
        out_specs=[
            full((NUM_GRAPHS, 1)),
            full((NUM_GRAPHS, 1)),
        ],
        out_shape=[
            jax.ShapeDtypeStruct((NUM_GRAPHS, 1), jnp.float32),
            jax.ShapeDtypeStruct((NUM_GRAPHS, 1), jnp.float32),
        ],
    )(poolp, mW1, mb1.reshape(1, HID), bn_gamma.reshape(1, HID),
      bn_beta.reshape(1, HID), mW2, mb2.reshape(1, 1))


def kernel(x, edge_index, batch, W1, b1, W2, b2, W3, b3, gW1, gb1, gW2, gb2,
           mW1, mb1, bn_gamma, bn_beta, mW2, mb2):
    pad_e = E_PAD - N_EDGES
    srcf = jnp.concatenate([edge_index[0], jnp.full((pad_e,), DUMMY, jnp.int32)])
    dstf = jnp.concatenate([edge_index[1], jnp.full((pad_e,), DUMMY, jnp.int32)])
    srcp = srcf.reshape(NW, STEPS, CH)
    dstp = dstf.reshape(NW, STEPS, CH)
    src2 = srcf.reshape(NS, NGRP, G, CH)
    dst2 = dstf.reshape(NS, NGRP, G, CH)
    xp = jnp.pad(x, ((0, NPAD - N_NODES), (0, 0)))
    batchp = jnp.concatenate(
        [batch, jnp.full((NPAD - N_NODES,), NUM_GRAPHS, jnp.int32)])
    ninf_tab = jnp.full((GP, HID), -jnp.inf, jnp.float32)

    degp = _deg_call(dstp)
    g, dinv = _embed_call(xp, degp.reshape(NC, NPAD, 1),
                          W1, b1, W2, b2, W3, b3)
    aggp = _main_call(src2, dst2, g)
    h2 = _gcn_call(aggp, dinv, gW1, gb1, gW2, gb2)
    poolp = _pool_call(h2, batchp, ninf_tab)
    out, sig = _head_call(poolp, mW1, mb1, bn_gamma, bn_beta, mW2, mb2)
    return (out, sig)


# speedup vs baseline: 1.1986x; 1.1986x over previous
import functools

import jax
import jax.numpy as jnp
from jax import lax
from jax.experimental import pallas as pl
from jax.experimental.pallas import tpu as pltpu
from jax.experimental.pallas import tpu_sc as plsc

N_NODES = 50000
N_EDGES = 1600000
IN_CH = 128
HID = 32
NUM_GRAPHS = 512
BN_EPS = 1e-5

NC = 2
NS = 16
NW = NC * NS
CH = 128
NBUF = 6
G = 36
NGRP = 22
STEPS_T = NGRP * G
E_PAD = STEPS_T * CH * NS
STEPS = E_PAD // NW // CH

NT = 1664
NPAD = NT * NW
GP = 520
DUMMY = N_NODES
WB = NPAD // NS


def _sc_deg_body(dst_hbm, out_hbm, dst_v, ones_v, zrow_v, deg_sh, dsem):
    c = lax.axis_index("c")
    s = lax.axis_index("s")
    w = c * NS + s
    pltpu.sync_copy(dst_hbm.at[w], dst_v)
    for i in range(CH // 16):
        ones_v[pl.ds(i * 16, 16)] = jnp.ones((16,), jnp.float32)

    def zbody(i, carry):
        zrow_v[pl.ds(i * 16, 16)] = jnp.zeros((16,), jnp.float32)
        return carry

    lax.fori_loop(0, WB // 16, zbody, 0)
    pltpu.sync_copy(zrow_v, deg_sh.at[pl.ds(s * WB, WB)])
    plsc.subcore_barrier()

    def body(q, carry):
        ds_ = [pltpu.async_copy(ones_v, deg_sh.at[dst_v.at[q * 8 + b]],
                                dsem, add=True) for b in range(8)]
        for d in ds_:
            d.wait()
        return carry

    lax.fori_loop(0, STEPS // 8, body, 0)
    tail = [pltpu.async_copy(ones_v, deg_sh.at[dst_v.at[(STEPS // 8) * 8 + b]],
                             dsem, add=True) for b in range(STEPS % 8)]
    for d in tail:
        d.wait()
    plsc.subcore_barrier()
    pltpu.sync_copy(deg_sh.at[pl.ds(s * WB, WB)],
                    out_hbm.at[pl.ds(c * NPAD + s * WB, WB)])


_deg_call = pl.kernel(
    _sc_deg_body,
    out_type=jax.ShapeDtypeStruct((NC * NPAD,), jnp.float32),
    mesh=plsc.VectorSubcoreMesh(core_axis_name="c", subcore_axis_name="s"),
    scratch_types=[
        pltpu.VMEM((STEPS, CH), jnp.int32),
        pltpu.VMEM((CH,), jnp.float32),
        pltpu.VMEM((WB,), jnp.float32),
        pltpu.VMEM_SHARED((NPAD,), jnp.float32),
        pltpu.SemaphoreType.DMA,
    ],
    compiler_params=pltpu.CompilerParams(use_tc_tiling_on_sc=False),
)


HH = HID // 2


def _sc_main_body(src_hbm, dst_hbm, g_hbm, out_hbm, src_v, dst_v, rows_v,
                  agg_sh, g_sh, gsem, ssem):
    c = lax.axis_index("c")
    s = lax.axis_index("s")
    pltpu.sync_copy(g_hbm.at[c, pl.ds(s * WB, WB)], g_sh.at[pl.ds(s * WB, WB)])
    z16 = jnp.zeros((16,), jnp.float32)
    zbuf = rows_v.at[0]
    for r in range(CH):
        zbuf[r, 0:16] = z16
    base = s * WB
    for k in range(WB // CH):
        pltpu.sync_copy(zbuf, agg_sh.at[pl.ds(base + k * CH, CH)])
    plsc.subcore_barrier()

    def grp(gi, carry):
        pltpu.sync_copy(src_hbm.at[s, gi], src_v)
        pltpu.sync_copy(dst_hbm.at[s, gi], dst_v)

        def octet(q, carry2):
            t0 = q * NBUF
            gd, sd = [], []
            for b in range(NBUF):
                gd.append(pltpu.async_copy(
                    g_sh.at[src_v.at[t0 + b]], rows_v.at[b], gsem))
            for d in gd:
                d.wait()
            for b in range(NBUF):
                sd.append(pltpu.async_copy(
                    rows_v.at[b], agg_sh.at[dst_v.at[t0 + b]], ssem, add=True))
            for d in sd:
                d.wait()
            return carry2

        lax.fori_loop(0, G // NBUF, octet, 0)
        return carry

    lax.fori_loop(0, NGRP, grp, 0)
    plsc.subcore_barrier()
    pltpu.sync_copy(agg_sh.at[pl.ds(s * WB, WB)], out_hbm.at[c, pl.ds(s * WB, WB)])


_main_call = pl.kernel(
    _sc_main_body,
    out_type=jax.ShapeDtypeStruct((NC, NPAD, HH), jnp.float32),
    mesh=plsc.VectorSubcoreMesh(core_axis_name="c", subcore_axis_name="s"),
    scratch_types=[
        pltpu.VMEM((G, CH), jnp.int32),
        pltpu.VMEM((G, CH), jnp.int32),
        pltpu.VMEM((NBUF, CH, HH), jnp.float32),
        pltpu.VMEM_SHARED((NPAD, HH), jnp.float32),
        pltpu.VMEM_SHARED((NPAD, HH), jnp.float32),
        pltpu.SemaphoreType.DMA,
        pltpu.SemaphoreType.DMA,
    ],
    compiler_params=pltpu.CompilerParams(use_tc_tiling_on_sc=False),
)


def _sc_pool_body(h2_hbm, batch_hbm, ninf_hbm, out_hbm, h2_v, b_v, pool_v):
    c = lax.axis_index("c")
    s = lax.axis_index("s")
    w = c * NS + s
    base = w * NT
    pltpu.sync_copy(h2_hbm.at[pl.ds(base, NT)], h2_v)
    pltpu.sync_copy(batch_hbm.at[pl.ds(base, NT)], b_v)
    pltpu.sync_copy(ninf_hbm, pool_v)
    iota = lax.iota(jnp.int32, 16)

    def body(n, carry):
        nspl = jnp.full((16,), n, jnp.int32)
        b = plsc.load_gather(b_v, [nspl])
        for half in (0, 16):
            col = iota + half
            rowv = plsc.load_gather(h2_v, [nspl, col])
            cur = plsc.load_gather(pool_v, [b, col])
            plsc.store_scatter(pool_v, [b, col], jnp.maximum(cur, rowv))
        return carry

    lax.fori_loop(0, NT, body, 0)
    pltpu.sync_copy(pool_v, out_hbm.at[w])


_pool_call = pl.kernel(
    _sc_pool_body,
    out_type=jax.ShapeDtypeStruct((NW, GP, HID), jnp.float32),
    mesh=plsc.VectorSubcoreMesh(core_axis_name="c", subcore_axis_name="s"),
    scratch_types=[
        pltpu.VMEM((NT, HID), jnp.float32),
        pltpu.VMEM((NT,), jnp.int32),
        pltpu.VMEM((GP, HID), jnp.float32),
    ],
    compiler_params=pltpu.CompilerParams(use_tc_tiling_on_sc=False,
                                         needs_layout_passes=False),
)


BLK = 1664
BLK_E = 1000


def _tc_embed_body(x_ref, w1, b1, w2, b2, w3, b3, h_ref):
    h = jnp.maximum(jnp.dot(x_ref[...], w1[...],
                            preferred_element_type=jnp.float32) + b1[...], 0.0)
    h = jnp.maximum(jnp.dot(h, w2[...],
                            preferred_element_type=jnp.float32) + b2[...], 0.0)
    h = jnp.dot(h, w3[...], preferred_element_type=jnp.float32) + b3[...]
    h_ref[0] = h[:, :HH]
    h_ref[1] = h[:, HH:]


def _embed_call(x, W1, b1, W2, b2, W3, b3):
    full = lambda shape: pl.BlockSpec(shape, lambda i: (0,) * len(shape))
    return pl.pallas_call(
        _tc_embed_body,
        grid=(N_NODES // BLK_E,),
        in_specs=[
            pl.BlockSpec((BLK_E, IN_CH), lambda i: (i, 0)),
            full((IN_CH, HID)), full((1, HID)),
            full((HID, HID)), full((1, HID)),
            full((HID, HID)), full((1, HID)),
        ],
        out_specs=pl.BlockSpec((NC, BLK_E, HH), lambda i: (0, i, 0)),
        out_shape=jax.ShapeDtypeStruct((NC, NPAD, HH), jnp.float32),
    )(x, W1, b1.reshape(1, HID), W2, b2.reshape(1, HID),
      W3, b3.reshape(1, HID))


def _dinv_of(degp_ref):
    deg = degp_ref[0] + degp_ref[1]
    return lax.rsqrt(jnp.maximum(deg, 1.0))[:, None]


def _tc_scale_body(h_ref, degp_ref, g_ref):
    dinv = _dinv_of(degp_ref)
    g_ref[0] = h_ref[0] * dinv
    g_ref[1] = h_ref[1] * dinv


def _scale_call(h, degp):
    return pl.pallas_call(
        _tc_scale_body,
        grid=(NPAD // BLK,),
        in_specs=[
            pl.BlockSpec((NC, BLK, HH), lambda i: (0, i, 0)),
            pl.BlockSpec((NC, BLK), lambda i: (0, i)),
        ],
        out_specs=pl.BlockSpec((NC, BLK, HH), lambda i: (0, i, 0)),
        out_shape=jax.ShapeDtypeStruct((NC, NPAD, HH), jnp.float32),
    )(h, degp)


def _leaky(x):
    return jnp.where(x >= 0.0, x, 0.01 * x)


def _tc_gcn_body(aggp_ref, degp_ref, w1, b1, w2, b2, h2_ref):
    dinv = _dinv_of(degp_ref)
    a = jnp.concatenate([aggp_ref[0], aggp_ref[1]], axis=1) * dinv
    t = _leaky(jnp.dot(a, w1[...], preferred_element_type=jnp.float32) + b1[...])
    h2_ref[...] = _leaky(jnp.dot(t, w2[...],
                                 preferred_element_type=jnp.float32) + b2[...])


def _gcn_call(aggp, degp, gW1, gb1, gW2, gb2):
    full = lambda shape: pl.BlockSpec(shape, lambda i: (0,) * len(shape))
    return pl.pallas_call(
        _tc_gcn_body,
        grid=(NPAD // BLK,),
        in_specs=[
            pl.BlockSpec((NC, BLK, HH), lambda i: (0, i, 0)),
            pl.BlockSpec((NC, BLK), lambda i: (0, i)),
            full((HID, HID)), full((1, HID)),
            full((HID, HID)), full((1, HID)),
        ],
        out_specs=pl.BlockSpec((BLK, HID), lambda i: (i, 0)),
        out_shape=jax.ShapeDtypeStruct((NPAD, HID), jnp.float32),
    )(aggp, degp, gW1, gb1.reshape(1, HID), gW2, gb2.reshape(1, HID))


def _tc_head_body(poolp_ref, mw1, mb1, gam, bet, mw2, mb2, out_ref, sig_ref):
    pm = jnp.max(poolp_ref[...], axis=0)[:NUM_GRAPHS]
    pooled = jnp.where(jnp.isfinite(pm), pm, 0.0)
    z = jnp.dot(pooled, mw1[...], preferred_element_type=jnp.float32) + mb1[...]
    z = (z / jnp.sqrt(1.0 + BN_EPS)) * gam[...] + bet[...]
    z = _leaky(z)
    o = jnp.dot(z, mw2[...], preferred_element_type=jnp.float32) + mb2[...]
    out_ref[...] = o
    sig_ref[...] = jax.nn.sigmoid(o)


def _head_call(poolp, mW1, mb1, bn_gamma, bn_beta, mW2, mb2):
    full = lambda shape: pl.BlockSpec(shape, lambda: (0,) * len(shape))
    return pl.pallas_call(
        _tc_head_body,
        in_specs=[
            full((NW, GP, HID)),
            full((HID, HID)), full((1, HID)),
            full((1, HID)), full((1, HID)),
            full((HID, 1)), full((1, 1)),
        ],
        out_specs=[
            full((NUM_GRAPHS, 1)),
            full((NUM_GRAPHS, 1)),
        ],
        out_shape=[
            jax.ShapeDtypeStruct((NUM_GRAPHS, 1), jnp.float32),
            jax.ShapeDtypeStruct((NUM_GRAPHS, 1), jnp.float32),
        ],
    )(poolp, mW1, mb1.reshape(1, HID), bn_gamma.reshape(1, HID),
      bn_beta.reshape(1, HID), mW2, mb2.reshape(1, 1))


def kernel(x, edge_index, batch, W1, b1, W2, b2, W3, b3, gW1, gb1, gW2, gb2,
           mW1, mb1, bn_gamma, bn_beta, mW2, mb2):
    pad_e = E_PAD - N_EDGES
    srcf = jnp.concatenate([edge_index[0], jnp.full((pad_e,), DUMMY, jnp.int32)])
    dstf = jnp.concatenate([edge_index[1], jnp.full((pad_e,), DUMMY, jnp.int32)])
    srcp = srcf.reshape(NW, STEPS, CH)
    dstp = dstf.reshape(NW, STEPS, CH)
    src2 = srcf.reshape(NS, NGRP, G, CH)
    dst2 = dstf.reshape(NS, NGRP, G, CH)
    batchp = jnp.concatenate(
        [batch, jnp.full((NPAD - N_NODES,), NUM_GRAPHS, jnp.int32)])
    ninf_tab = jnp.full((GP, HID), -jnp.inf, jnp.float32)

    degp = _deg_call(dstp).reshape(NC, NPAD)
    h = _embed_call(x, W1, b1, W2, b2, W3, b3)
    g = _scale_call(h, degp)
    aggp = _main_call(src2, dst2, g)
    h2 = _gcn_call(aggp, degp, gW1, gb1, gW2, gb2)
    poolp = _pool_call(h2, batchp, ninf_tab)
    out, sig = _head_call(poolp, mW1, mb1, bn_gamma, bn_beta, mW2, mb2)
    return (out, sig)


# speedup vs baseline: 1.2420x; 1.0363x over previous
import functools

import jax
import jax.numpy as jnp
from jax import lax
from jax.experimental import pallas as pl
from jax.experimental.pallas import tpu as pltpu
from jax.experimental.pallas import tpu_sc as plsc

N_NODES = 50000
N_EDGES = 1600000
IN_CH = 128
HID = 32
NUM_GRAPHS = 512
BN_EPS = 1e-5

NC = 2
NS = 16
NW = NC * NS
CH = 128
NBUF = 6
G = 24
NGRP = 33
STEPS_T = NGRP * G
E_PAD = STEPS_T * CH * NS
STEPS = E_PAD // NW // CH
EI_ROWS = N_EDGES // CH

NT = 1664
NPAD = NT * NW
GP = 520
DUMMY = N_NODES
WB = NPAD // NS


def _sc_deg_body(dst_hbm, out_hbm, dst_v, ones_v, zrow_v, deg_sh, dsem):
    c = lax.axis_index("c")
    s = lax.axis_index("s")
    w = c * NS + s
    pltpu.sync_copy(dst_hbm.at[w], dst_v)
    for i in range(CH // 16):
        ones_v[pl.ds(i * 16, 16)] = jnp.ones((16,), jnp.float32)

    def zbody(i, carry):
        zrow_v[pl.ds(i * 16, 16)] = jnp.zeros((16,), jnp.float32)
        return carry

    lax.fori_loop(0, WB // 16, zbody, 0)
    pltpu.sync_copy(zrow_v, deg_sh.at[pl.ds(s * WB, WB)])
    plsc.subcore_barrier()

    def body(q, carry):
        ds_ = [pltpu.async_copy(ones_v, deg_sh.at[dst_v.at[q * 8 + b]],
                                dsem, add=True) for b in range(8)]
        for d in ds_:
            d.wait()
        return carry

    lax.fori_loop(0, STEPS // 8, body, 0)
    tail = [pltpu.async_copy(ones_v, deg_sh.at[dst_v.at[(STEPS // 8) * 8 + b]],
                             dsem, add=True) for b in range(STEPS % 8)]
    for d in tail:
        d.wait()
    plsc.subcore_barrier()
    pltpu.sync_copy(deg_sh.at[pl.ds(s * WB, WB)],
                    out_hbm.at[pl.ds(c * NPAD + s * WB, WB)])


_deg_call = pl.kernel(
    _sc_deg_body,
    out_type=jax.ShapeDtypeStruct((NC * NPAD,), jnp.float32),
    mesh=plsc.VectorSubcoreMesh(core_axis_name="c", subcore_axis_name="s"),
    scratch_types=[
        pltpu.VMEM((STEPS, CH), jnp.int32),
        pltpu.VMEM((CH,), jnp.float32),
        pltpu.VMEM((WB,), jnp.float32),
        pltpu.VMEM_SHARED((NPAD,), jnp.float32),
        pltpu.SemaphoreType.DMA,
    ],
    compiler_params=pltpu.CompilerParams(use_tc_tiling_on_sc=False),
)


HH = HID // 2


def _sc_main_body(src_hbm, dst_hbm, g_hbm, out_hbm, src_v, dst_v, rows_v,
                  agg_sh, g_sh, gsem, ssem):
    c = lax.axis_index("c")
    s = lax.axis_index("s")
    pltpu.sync_copy(g_hbm.at[c, pl.ds(s * WB, WB)], g_sh.at[pl.ds(s * WB, WB)])
    z16 = jnp.zeros((16,), jnp.float32)
    zbuf = rows_v.at[0]
    for r in range(CH):
        zbuf[r, 0:16] = z16
    base = s * WB
    for k in range(WB // CH):
        pltpu.sync_copy(zbuf, agg_sh.at[pl.ds(base + k * CH, CH)])
    plsc.subcore_barrier()

    def grp(gi, carry):
        pltpu.sync_copy(src_hbm.at[s, pl.ds(gi * G, G)], src_v)
        pltpu.sync_copy(dst_hbm.at[s, pl.ds(gi * G, G)], dst_v)

        def octet(q, carry2):
            t0 = q * NBUF
            gd, sd = [], []
            for b in range(NBUF):
                gd.append(pltpu.async_copy(
                    g_sh.at[src_v.at[t0 + b]], rows_v.at[b], gsem))
            for d in gd:
                d.wait()
            for b in range(NBUF):
                sd.append(pltpu.async_copy(
                    rows_v.at[b], agg_sh.at[dst_v.at[t0 + b]], ssem, add=True))
            for d in sd:
                d.wait()
            return carry2

        lax.fori_loop(0, G // NBUF, octet, 0)
        return carry

    lax.fori_loop(0, NGRP, grp, 0)
    plsc.subcore_barrier()
    pltpu.sync_copy(agg_sh.at[pl.ds(s * WB, WB)], out_hbm.at[c, pl.ds(s * WB, WB)])


_main_call = pl.kernel(
    _sc_main_body,
    out_type=jax.ShapeDtypeStruct((NC, NPAD, HH), jnp.float32),
    mesh=plsc.VectorSubcoreMesh(core_axis_name="c", subcore_axis_name="s"),
    scratch_types=[
        pltpu.VMEM((G, CH), jnp.int32),
        pltpu.VMEM((G, CH), jnp.int32),
        pltpu.VMEM((NBUF, CH, HH), jnp.float32),
        pltpu.VMEM_SHARED((NPAD, HH), jnp.float32),
        pltpu.VMEM_SHARED((NPAD, HH), jnp.float32),
        pltpu.SemaphoreType.DMA,
        pltpu.SemaphoreType.DMA,
    ],
    compiler_params=pltpu.CompilerParams(use_tc_tiling_on_sc=False),
)


def _sc_pool_body(h2_hbm, batch_hbm, ninf_hbm, out_hbm, h2_v, b_v, pool_v):
    c = lax.axis_index("c")
    s = lax.axis_index("s")
    w = c * NS + s
    base = w * NT
    pltpu.sync_copy(h2_hbm.at[pl.ds(base, NT)], h2_v)
    pltpu.sync_copy(batch_hbm.at[pl.ds(base, NT)], b_v)
    pltpu.sync_copy(ninf_hbm, pool_v)
    iota = lax.iota(jnp.int32, 16)

    def body(n, carry):
        nspl = jnp.full((16,), n, jnp.int32)
        b = plsc.load_gather(b_v, [nspl])
        for half in (0, 16):
            col = iota + half
            rowv = plsc.load_gather(h2_v, [nspl, col])
            cur = plsc.load_gather(pool_v, [b, col])
            plsc.store_scatter(pool_v, [b, col], jnp.maximum(cur, rowv))
        return carry

    lax.fori_loop(0, NT, body, 0)
    pltpu.sync_copy(pool_v, out_hbm.at[w])


_pool_call = pl.kernel(
    _sc_pool_body,
    out_type=jax.ShapeDtypeStruct((NW, GP, HID), jnp.float32),
    mesh=plsc.VectorSubcoreMesh(core_axis_name="c", subcore_axis_name="s"),
    scratch_types=[
        pltpu.VMEM((NT, HID), jnp.float32),
        pltpu.VMEM((NT,), jnp.int32),
        pltpu.VMEM((GP, HID), jnp.float32),
    ],
    compiler_params=pltpu.CompilerParams(use_tc_tiling_on_sc=False,
                                         needs_layout_passes=False),
)


BLK = 1664
BLK_E = 1000


def _tc_edge_body(ei_ref, src_ref, dst_ref):
    s = pl.program_id(0)
    rows = jax.lax.broadcasted_iota(jnp.int32, (STEPS_T, CH), 0) + s * STEPS_T
    mask = rows < EI_ROWS
    src_ref[0] = jnp.where(mask, ei_ref[0], DUMMY)
    dst_ref[0] = jnp.where(mask, ei_ref[1], DUMMY)


def _edge_call(ei3):
    shp = jax.ShapeDtypeStruct((NS, STEPS_T, CH), jnp.int32)
    return pl.pallas_call(
        _tc_edge_body,
        grid=(NS,),
        in_specs=[pl.BlockSpec((2, STEPS_T, CH), lambda s: (0, s, 0))],
        out_specs=[pl.BlockSpec((1, STEPS_T, CH), lambda s: (s, 0, 0)),
                   pl.BlockSpec((1, STEPS_T, CH), lambda s: (s, 0, 0))],
        out_shape=[shp, shp],
    )(ei3)


def _tc_embed_body(x_ref, w1, b1, w2, b2, w3, b3, h_ref):
    h = jnp.maximum(jnp.dot(x_ref[...], w1[...],
                            preferred_element_type=jnp.float32) + b1[...], 0.0)
    h = jnp.maximum(jnp.dot(h, w2[...],
                            preferred_element_type=jnp.float32) + b2[...], 0.0)
    h = jnp.dot(h, w3[...], preferred_element_type=jnp.float32) + b3[...]
    h_ref[0] = h[:, :HH]
    h_ref[1] = h[:, HH:]


def _embed_call(x, W1, b1, W2, b2, W3, b3):
    full = lambda shape: pl.BlockSpec(shape, lambda i: (0,) * len(shape))
    return pl.pallas_call(
        _tc_embed_body,
        grid=(N_NODES // BLK_E,),
        in_specs=[
            pl.BlockSpec((BLK_E, IN_CH), lambda i: (i, 0)),
            full((IN_CH, HID)), full((1, HID)),
            full((HID, HID)), full((1, HID)),
            full((HID, HID)), full((1, HID)),
        ],
        out_specs=pl.BlockSpec((NC, BLK_E, HH), lambda i: (0, i, 0)),
        out_shape=jax.ShapeDtypeStruct((NC, NPAD, HH), jnp.float32),
    )(x, W1, b1.reshape(1, HID), W2, b2.reshape(1, HID),
      W3, b3.reshape(1, HID))


def _dinv_of(degp_ref):
    deg = degp_ref[0] + degp_ref[1]
    return lax.rsqrt(jnp.maximum(deg, 1.0))[:, None]


def _tc_scale_body(h_ref, degp_ref, g_ref):
    dinv = _dinv_of(degp_ref)
    g_ref[0] = h_ref[0] * dinv
    g_ref[1] = h_ref[1] * dinv


def _scale_call(h, degp):
    return pl.pallas_call(
        _tc_scale_body,
        grid=(NPAD // BLK,),
        in_specs=[
            pl.BlockSpec((NC, BLK, HH), lambda i: (0, i, 0)),
            pl.BlockSpec((NC, BLK), lambda i: (0, i)),
        ],
        out_specs=pl.BlockSpec((NC, BLK, HH), lambda i: (0, i, 0)),
        out_shape=jax.ShapeDtypeStruct((NC, NPAD, HH), jnp.float32),
    )(h, degp)


def _leaky(x):
    return jnp.where(x >= 0.0, x, 0.01 * x)


def _tc_gcn_body(aggp_ref, degp_ref, w1, b1, w2, b2, h2_ref):
    dinv = _dinv_of(degp_ref)
    a = jnp.concatenate([aggp_ref[0], aggp_ref[1]], axis=1) * dinv
    t = _leaky(jnp.dot(a, w1[...], preferred_element_type=jnp.float32) + b1[...])
    h2_ref[...] = _leaky(jnp.dot(t, w2[...],
                                 preferred_element_type=jnp.float32) + b2[...])


def _gcn_call(aggp, degp, gW1, gb1, gW2, gb2):
    full = lambda shape: pl.BlockSpec(shape, lambda i: (0,) * len(shape))
    return pl.pallas_call(
        _tc_gcn_body,
        grid=(NPAD // BLK,),
        in_specs=[
            pl.BlockSpec((NC, BLK, HH), lambda i: (0, i, 0)),
            pl.BlockSpec((NC, BLK), lambda i: (0, i)),
            full((HID, HID)), full((1, HID)),
            full((HID, HID)), full((1, HID)),
        ],
        out_specs=pl.BlockSpec((BLK, HID), lambda i: (i, 0)),
        out_shape=jax.ShapeDtypeStruct((NPAD, HID), jnp.float32),
    )(aggp, degp, gW1, gb1.reshape(1, HID), gW2, gb2.reshape(1, HID))


def _tc_head_body(poolp_ref, mw1, mb1, gam, bet, mw2, mb2, out_ref, sig_ref):
    pm = jnp.max(poolp_ref[...], axis=0)[:NUM_GRAPHS]
    pooled = jnp.where(jnp.isfinite(pm), pm, 0.0)
    z = jnp.dot(pooled, mw1[...], preferred_element_type=jnp.float32) + mb1[...]
    z = (z / jnp.sqrt(1.0 + BN_EPS)) * gam[...] + bet[...]
    z = _leaky(z)
    o = jnp.dot(z, mw2[...], preferred_element_type=jnp.float32) + mb2[...]
    out_ref[...] = o
    sig_ref[...] = jax.nn.sigmoid(o)


def _head_call(poolp, mW1, mb1, bn_gamma, bn_beta, mW2, mb2):
    full = lambda shape: pl.BlockSpec(shape, lambda: (0,) * len(shape))
    return pl.pallas_call(
        _tc_head_body,
        in_specs=[
            full((NW, GP, HID)),
            full((HID, HID)), full((1, HID)),
            full((1, HID)), full((1, HID)),
            full((HID, 1)), full((1, 1)),
        ],
        out_specs=[
            full((NUM_GRAPHS, 1)),
            full((NUM_GRAPHS, 1)),
        ],
        out_shape=[
            jax.ShapeDtypeStruct((NUM_GRAPHS, 1), jnp.float32),
            jax.ShapeDtypeStruct((NUM_GRAPHS, 1), jnp.float32),
        ],
    )(poolp, mW1, mb1.reshape(1, HID), bn_gamma.reshape(1, HID),
      bn_beta.reshape(1, HID), mW2, mb2.reshape(1, 1))


def kernel(x, edge_index, batch, W1, b1, W2, b2, W3, b3, gW1, gb1, gW2, gb2,
           mW1, mb1, bn_gamma, bn_beta, mW2, mb2):
    batchp = jnp.concatenate(
        [batch, jnp.full((NPAD - N_NODES,), NUM_GRAPHS, jnp.int32)])
    ninf_tab = jnp.full((GP, HID), -jnp.inf, jnp.float32)

    ei3 = edge_index.reshape(2, EI_ROWS, CH)
    srcp, dstp = _edge_call(ei3)
    degp = _deg_call(dstp.reshape(NW, STEPS, CH)).reshape(NC, NPAD)
    h = _embed_call(x, W1, b1, W2, b2, W3, b3)
    g = _scale_call(h, degp)
    aggp = _main_call(srcp, dstp, g)
    h2 = _gcn_call(aggp, degp, gW1, gb1, gW2, gb2)
    poolp = _pool_call(h2, batchp, ninf_tab)
    out, sig = _head_call(poolp, mW1, mb1, bn_gamma, bn_beta, mW2, mb2)
    return (out, sig)
